# baseline (speedup 1.0000x reference)
import functools

import jax
import jax.numpy as jnp
from jax import lax
from jax.experimental import pallas as pl
from jax.experimental.pallas import tpu as pltpu
from jax.experimental.pallas import tpu_sc as plsc

N = 10000
D = 128
NT = 32
EPT = 10000
NCH = 125
ECH = 80

RPT = N // 16
DEGW = 128

_MESH = plsc.VectorSubcoreMesh(core_axis_name="c", subcore_axis_name="s")


@functools.partial(
    pl.kernel,
    mesh=_MESH,
    out_type=jax.ShapeDtypeStruct((2, 16, RPT, DEGW), jnp.float32),
    scratch_types=[
        pltpu.VMEM((NCH, ECH), jnp.int32),
        pltpu.VMEM((ECH, DEGW), jnp.float32),
        pltpu.VMEM((25, DEGW), jnp.float32),
        pltpu.VMEM_SHARED((N, DEGW), jnp.float32),
        pltpu.SemaphoreType.DMA,
        pltpu.SemaphoreType.DMA,
    ],
)
def _deg_sc(dst_hbm, out_hbm, dst_v, ones_v, zbuf, acc, sa, sb):
    c = lax.axis_index("c")
    s = lax.axis_index("s")
    tid = s * 2 + c
    one16 = jnp.ones((16,), jnp.float32)
    zero16 = jnp.zeros((16,), jnp.float32)

    def fill(i, _):
        for k in range(DEGW // 16):
            ones_v[i, pl.ds(k * 16, 16)] = one16
        return 0

    lax.fori_loop(0, ECH, fill, 0)

    def zrow(i, _):
        for k in range(DEGW // 16):
            zbuf[i, pl.ds(k * 16, 16)] = zero16
        return 0

    lax.fori_loop(0, 25, zrow, 0)
    for j in range(RPT // 25):
        pltpu.sync_copy(zbuf, acc.at[pl.ds(s * RPT + j * 25, 25)])
    pltpu.sync_copy(dst_hbm.at[tid], dst_v)
    plsc.subcore_barrier()

    pltpu.async_copy(ones_v, acc.at[dst_v.at[0]], sa, add=True)

    def chunk(i, _):
        ci = 2 * i
        pltpu.async_copy(ones_v, acc.at[dst_v.at[ci + 1]], sb, add=True)

        @pl.when(i > 0)
        def _():
            pltpu.make_async_copy(ones_v, acc.at[dst_v.at[0]], sa).wait()

        @pl.when(i < NCH // 2 - 1)
        def _():
            pltpu.async_copy(ones_v, acc.at[dst_v.at[ci + 2]], sa, add=True)
        pltpu.make_async_copy(ones_v, acc.at[dst_v.at[0]], sb).wait()
        return 0

    lax.fori_loop(0, NCH // 2, chunk, 0)
    pltpu.async_copy(ones_v, acc.at[dst_v.at[NCH - 1]], sb, add=True)
    pltpu.make_async_copy(ones_v, acc.at[dst_v.at[0]], sa).wait()
    pltpu.make_async_copy(ones_v, acc.at[dst_v.at[0]], sb).wait()
    plsc.subcore_barrier()
    pltpu.sync_copy(acc.at[pl.ds(s * RPT, RPT)], out_hbm.at[c].at[s])


@functools.partial(
    pl.kernel,
    mesh=_MESH,
    out_type=jax.ShapeDtypeStruct((2, 16, RPT, D), jnp.float32),
    scratch_types=[
        pltpu.VMEM((EPT // 2 + 120,), jnp.int32),
        pltpu.VMEM((ECH,), jnp.int32),
        pltpu.VMEM((ECH,), jnp.int32),
        pltpu.VMEM((ECH,), jnp.int32),
        pltpu.VMEM((ECH,), jnp.int32),
        pltpu.VMEM((ECH,), jnp.int32),
        pltpu.VMEM((ECH,), jnp.int32),
        pltpu.VMEM((ECH,), jnp.int32),
        pltpu.VMEM((ECH,), jnp.int32),
        pltpu.VMEM((ECH, D), jnp.float32),
        pltpu.VMEM((ECH, D), jnp.float32),
        pltpu.VMEM((ECH, D), jnp.float32),
        pltpu.VMEM((ECH, D), jnp.float32),
        pltpu.VMEM_SHARED((N, D), jnp.float32),
        pltpu.SemaphoreType.DMA,
        pltpu.SemaphoreType.DMA,
        pltpu.SemaphoreType.DMA,
        pltpu.SemaphoreType.DMA,
        pltpu.SemaphoreType.DMA,
        pltpu.SemaphoreType.DMA,
        pltpu.SemaphoreType.DMA,
        pltpu.SemaphoreType.DMA,
    ],
)
def _spmm_sc(sd_hbm, x_hbm, out_hbm, sd_v,
             s0, s1, s2, s3, d0, d1, d2, d3,
             r0, r1, r2, r3, acc,
             ga0, ga1, ga2, ga3, sa0, sa1, sa2, sa3):
    c = lax.axis_index("c")
    s = lax.axis_index("s")
    tid = s * 2 + c
    zero16 = jnp.zeros((16,), jnp.float32)

    def zrow(i, _):
        for k in range(D // 16):
            r0[i, pl.ds(k * 16, 16)] = zero16
        return 0

    lax.fori_loop(0, ECH, zrow, 0)
    for j in range(7):
        pltpu.sync_copy(r0, acc.at[pl.ds(s * RPT + j * ECH, ECH)])
    pltpu.sync_copy(r0.at[pl.ds(0, 65)], acc.at[pl.ds(s * RPT + 560, 65)])
    pltpu.sync_copy(sd_hbm.at[tid, 0], sd_v)
    plsc.subcore_barrier()

    SB = (s0, s1, s2, s3)
    DB = (d0, d1, d2, d3)
    RW = (r0, r1, r2, r3)
    GS = (ga0, ga1, ga2, ga3)
    SS = (sa0, sa1, sa2, sa3)
    HCH = 64

    def unpack(ci, u):
        base = jnp.where(ci < HCH, ci, ci - HCH) * ECH
        for k in range(ECH // 16):
            v = sd_v[pl.ds(base + k * 16, 16)]
            SB[u][pl.ds(k * 16, 16)] = jnp.bitwise_and(v, 0xFFFF)
            DB[u][pl.ds(k * 16, 16)] = lax.shift_right_logical(v, 16)

    def gstart(u):
        pltpu.async_copy(x_hbm.at[SB[u]], RW[u], GS[u])

    def gwait(u):
        pltpu.make_async_copy(x_hbm.at[SB[u]], RW[u], GS[u]).wait()

    def sstart(u):
        pltpu.async_copy(RW[u], acc.at[DB[u]], SS[u], add=True)

    def swait(u):
        pltpu.make_async_copy(RW[u], acc.at[DB[u]], SS[u]).wait()

    unpack(0, 0)
    gstart(0)
    unpack(1, 1)
    gstart(1)
    NQ = 31

    def quad(i, _):
        for u in range(4):
            t = 4 * i + u
            pu = (u + 2) % 4

            if u < 2:
                @pl.when(i > 0)
                def _():
                    swait(pu)
            else:
                swait(pu)

            if u == 2:
                @pl.when(i == 15)
                def _():
                    pltpu.sync_copy(sd_hbm.at[tid, 1], sd_v)

            if u == 3:
                @pl.when(i < NQ - 1)
                def _():
                    unpack(t + 2, pu)
                    gstart(pu)
            else:
                unpack(t + 2, pu)
                gstart(pu)

            gwait(u)
            sstart(u)
        return 0

    lax.fori_loop(0, NQ, quad, 0)
    gwait(0)
    sstart(0)
    swait(2)
    swait(3)
    swait(0)
    plsc.subcore_barrier()
    pltpu.sync_copy(acc.at[pl.ds(s * RPT, RPT)], out_hbm.at[c].at[s])


R = 1000
_SELU_ALPHA = 1.6732632423543772
_SELU_SCALE = 1.0507009873554805


def _dinv_of(deg_ref):
    dg = deg_ref[:, 0:1] + deg_ref[:, 1:2]
    return jnp.where(dg > 0.0, lax.rsqrt(dg), 0.0)


def _dninv_of(deg_ref):
    dg = deg_ref[:, 0:1] + deg_ref[:, 1:2]
    return 1.0 / jnp.maximum(dg, 1.0)


def _matt(a, w):
    return lax.dot_general(a, w, (((1,), (1,)), ((), ())),
                           preferred_element_type=jnp.float32)


_deg_spec = pl.BlockSpec((R, 2), lambda i: (i, 0))
_p_spec = pl.BlockSpec((2, R, D), lambda i: (0, i, 0))
_m_spec = pl.BlockSpec((R, D), lambda i: (i, 0))
_w_spec = pl.BlockSpec((D, D), lambda i: (0, 0))
_b_spec = pl.BlockSpec((1, D), lambda i: (0, 0))
_m_shape = jax.ShapeDtypeStruct((N, D), jnp.float32)


def _t0_body(deg_ref, x_ref, g0_ref):
    g0_ref[...] = x_ref[...] * _dinv_of(deg_ref)


_t0 = pl.pallas_call(
    _t0_body, grid=(N // R,),
    in_specs=[_deg_spec, _m_spec],
    out_specs=_m_spec, out_shape=_m_shape)


def _t1_body(deg_ref, p_ref, x_ref, h1_ref, g1_ref):
    dinv = _dinv_of(deg_ref)
    h1 = (p_ref[0] + p_ref[1]) * dinv + x_ref[...]
    h1_ref[...] = h1
    g1_ref[...] = h1 * dinv


_t1 = pl.pallas_call(
    _t1_body, grid=(N // R,),
    in_specs=[_deg_spec, _p_spec, _m_spec],
    out_specs=(_m_spec, _m_spec), out_shape=(_m_shape, _m_shape))


def _t2_body(deg_ref, p_ref, h1_ref, h_ref):
    h_ref[...] = (p_ref[0] + p_ref[1]) * _dinv_of(deg_ref) + h1_ref[...]


_t2 = pl.pallas_call(
    _t2_body, grid=(N // R,),
    in_specs=[_deg_spec, _p_spec, _m_spec],
    out_specs=_m_spec, out_shape=_m_shape)


def _t3_body(deg_ref, p_ref, h_ref, w1l_ref, b1_ref, w1r_ref, h2_ref):
    mean1 = (p_ref[0] + p_ref[1]) * _dninv_of(deg_ref)
    z = _matt(mean1, w1l_ref[...]) + b1_ref[...] + _matt(h_ref[...],
                                                         w1r_ref[...])
    h2_ref[...] = _SELU_SCALE * jnp.where(
        z > 0.0, z, _SELU_ALPHA * (jnp.exp(z) - 1.0))


_t3 = pl.pallas_call(
    _t3_body, grid=(N // R,),
    in_specs=[_deg_spec, _p_spec, _m_spec, _w_spec, _b_spec, _w_spec],
    out_specs=_m_spec, out_shape=_m_shape)


def _t4_body(deg_ref, p_ref, h2_ref, w2l_ref, b2_ref, w2r_ref, out_ref):
    mean2 = (p_ref[0] + p_ref[1]) * _dninv_of(deg_ref)
    z = _matt(mean2, w2l_ref[...]) + b2_ref[...] + _matt(h2_ref[...],
                                                         w2r_ref[...])
    z = z - jnp.max(z, axis=1, keepdims=True)
    ez = jnp.exp(z)
    out_ref[...] = ez / jnp.sum(ez, axis=1, keepdims=True)


_t4 = pl.pallas_call(
    _t4_body, grid=(N // R,),
    in_specs=[_deg_spec, _p_spec, _m_spec, _w_spec, _b_spec, _w_spec],
    out_specs=_m_spec, out_shape=_m_shape)


def kernel(x, edge_index, W1_l, b1, W1_r, W2_l, b2, W2_r):
    sdf = (edge_index[0] | (edge_index[1] << 16)).reshape(NT, EPT)
    sd = jnp.pad(sdf, ((0, 0), (0, 240))).reshape(NT, 2, EPT // 2 + 120)
    dstd = edge_index[1].reshape(NT, NCH, ECH)
    b1r = b1.reshape(1, D)
    b2r = b2.reshape(1, D)

    def spmm(xin):
        return _spmm_sc(sd, xin).reshape(2, N, D)

    degt = _deg_sc(dstd)[:, :, :, 0].reshape(2, N).T
    g0 = _t0(degt, x)
    p1 = spmm(g0)
    h1, g1 = _t1(degt, p1, x)
    p2 = spmm(g1)
    h = _t2(degt, p2, h1)
    p3 = spmm(h)
    h2 = _t3(degt, p3, h, W1_l, b1r, W1_r)
    p4 = spmm(h2)
    out = _t4(degt, p4, h2, W2_l, b2r, W2_r)
    return out


# speedup vs baseline: 1.0403x; 1.0403x over previous
import functools

import jax
import jax.numpy as jnp
from jax import lax
from jax.experimental import pallas as pl
from jax.experimental.pallas import tpu as pltpu
from jax.experimental.pallas import tpu_sc as plsc

N = 10000
D = 128
NT = 32
EPT = 10000
NCH = 125
ECH = 80

RPT = N // 16
DEGW = 128

_MESH = plsc.VectorSubcoreMesh(core_axis_name="c", subcore_axis_name="s")


@functools.partial(
    pl.kernel,
    mesh=_MESH,
    out_type=jax.ShapeDtypeStruct((2, 16, RPT, DEGW), jnp.float32),
    scratch_types=[
        pltpu.VMEM((NCH, ECH), jnp.int32),
        pltpu.VMEM((ECH, DEGW), jnp.float32),
        pltpu.VMEM((25, DEGW), jnp.float32),
        pltpu.VMEM_SHARED((N, DEGW), jnp.float32),
        pltpu.SemaphoreType.DMA,
        pltpu.SemaphoreType.DMA,
    ],
)
def _deg_sc(dst_hbm, out_hbm, dst_v, ones_v, zbuf, acc, sa, sb):
    c = lax.axis_index("c")
    s = lax.axis_index("s")
    tid = s * 2 + c
    one16 = jnp.ones((16,), jnp.float32)
    zero16 = jnp.zeros((16,), jnp.float32)

    def fill(i, _):
        for k in range(DEGW // 16):
            ones_v[i, pl.ds(k * 16, 16)] = one16
        return 0

    lax.fori_loop(0, ECH, fill, 0)

    def zrow(i, _):
        for k in range(DEGW // 16):
            zbuf[i, pl.ds(k * 16, 16)] = zero16
        return 0

    lax.fori_loop(0, 25, zrow, 0)
    for j in range(RPT // 25):
        pltpu.sync_copy(zbuf, acc.at[pl.ds(s * RPT + j * 25, 25)])
    pltpu.sync_copy(dst_hbm.at[tid], dst_v)
    plsc.subcore_barrier()

    pltpu.async_copy(ones_v, acc.at[dst_v.at[0]], sa, add=True)

    def chunk(i, _):
        ci = 2 * i
        pltpu.async_copy(ones_v, acc.at[dst_v.at[ci + 1]], sb, add=True)

        @pl.when(i > 0)
        def _():
            pltpu.make_async_copy(ones_v, acc.at[dst_v.at[0]], sa).wait()

        @pl.when(i < NCH // 2 - 1)
        def _():
            pltpu.async_copy(ones_v, acc.at[dst_v.at[ci + 2]], sa, add=True)
        pltpu.make_async_copy(ones_v, acc.at[dst_v.at[0]], sb).wait()
        return 0

    lax.fori_loop(0, NCH // 2, chunk, 0)
    pltpu.async_copy(ones_v, acc.at[dst_v.at[NCH - 1]], sb, add=True)
    pltpu.make_async_copy(ones_v, acc.at[dst_v.at[0]], sa).wait()
    pltpu.make_async_copy(ones_v, acc.at[dst_v.at[0]], sb).wait()
    plsc.subcore_barrier()
    pltpu.sync_copy(acc.at[pl.ds(s * RPT, RPT)], out_hbm.at[c].at[s])


@functools.partial(
    pl.kernel,
    mesh=_MESH,
    out_type=jax.ShapeDtypeStruct((2, 16, RPT, D), jnp.float32),
    scratch_types=[
        pltpu.VMEM((EPT,), jnp.int32),
        pltpu.VMEM((ECH,), jnp.int32),
        pltpu.VMEM((ECH,), jnp.int32),
        pltpu.VMEM((ECH,), jnp.int32),
        pltpu.VMEM((ECH,), jnp.int32),
        pltpu.VMEM((ECH,), jnp.int32),
        pltpu.VMEM((ECH,), jnp.int32),
        pltpu.VMEM((ECH, D), jnp.float32),
        pltpu.VMEM((ECH, D), jnp.float32),
        pltpu.VMEM((ECH, D), jnp.float32),
        pltpu.VMEM((25, D), jnp.float32),
        pltpu.VMEM_SHARED((N, D), jnp.float32),
        pltpu.SemaphoreType.DMA,
        pltpu.SemaphoreType.DMA,
        pltpu.SemaphoreType.DMA,
        pltpu.SemaphoreType.DMA,
        pltpu.SemaphoreType.DMA,
        pltpu.SemaphoreType.DMA,
    ],
)
def _spmm_sc(sd_hbm, x_hbm, out_hbm, sd_v,
             s0, s1, s2, d0, d1, d2,
             r0, r1, r2, zbuf, acc,
             ga0, ga1, ga2, sa0, sa1, sa2):
    c = lax.axis_index("c")
    s = lax.axis_index("s")
    tid = s * 2 + c
    zero16 = jnp.zeros((16,), jnp.float32)

    def zrow(i, _):
        for k in range(D // 16):
            zbuf[i, pl.ds(k * 16, 16)] = zero16
        return 0

    lax.fori_loop(0, 25, zrow, 0)
    for j in range(RPT // 25):
        pltpu.sync_copy(zbuf, acc.at[pl.ds(s * RPT + j * 25, 25)])
    pltpu.sync_copy(sd_hbm.at[tid], sd_v)
    plsc.subcore_barrier()

    SB = (s0, s1, s2)
    DB = (d0, d1, d2)
    RW = (r0, r1, r2)
    GS = (ga0, ga1, ga2)
    SS = (sa0, sa1, sa2)

    def unpack(ci, u):
        for k in range(ECH // 16):
            v = sd_v[pl.ds(ci * ECH + k * 16, 16)]
            SB[u][pl.ds(k * 16, 16)] = jnp.bitwise_and(v, 0xFFFF)
            DB[u][pl.ds(k * 16, 16)] = lax.shift_right_logical(v, 16)

    def gstart(u):
        pltpu.async_copy(x_hbm.at[SB[u]], RW[u], GS[u])

    def gwait(u):
        pltpu.make_async_copy(x_hbm.at[SB[u]], RW[u], GS[u]).wait()

    def sstart(u):
        pltpu.async_copy(RW[u], acc.at[DB[u]], SS[u], add=True)

    def swait(u):
        pltpu.make_async_copy(RW[u], acc.at[DB[u]], SS[u]).wait()

    unpack(0, 0)
    gstart(0)
    unpack(1, 1)
    gstart(1)
    NQ = 41

    def triple(i, _):
        for u in range(3):
            t = 3 * i + u
            pu = (u + 2) % 3

            if u == 0:
                @pl.when(i > 0)
                def _():
                    swait(pu)
            else:
                swait(pu)

            unpack(t + 2, pu)
            gstart(pu)
            gwait(u)
            sstart(u)
        return 0

    lax.fori_loop(0, NQ, triple, 0)
    gwait(0)
    sstart(0)
    gwait(1)
    sstart(1)
    swait(2)
    swait(0)
    swait(1)
    plsc.subcore_barrier()
    pltpu.sync_copy(acc.at[pl.ds(s * RPT, RPT)], out_hbm.at[c].at[s])


R = 1000
_SELU_ALPHA = 1.6732632423543772
_SELU_SCALE = 1.0507009873554805


def _dinv_of(deg_ref):
    dg = deg_ref[:, 0:1] + deg_ref[:, 1:2]
    return jnp.where(dg > 0.0, lax.rsqrt(dg), 0.0)


def _dninv_of(deg_ref):
    dg = deg_ref[:, 0:1] + deg_ref[:, 1:2]
    return 1.0 / jnp.maximum(dg, 1.0)


def _matt(a, w):
    return lax.dot_general(a, w, (((1,), (1,)), ((), ())),
                           preferred_element_type=jnp.float32)


_deg_spec = pl.BlockSpec((R, 2), lambda i: (i, 0))
_p_spec = pl.BlockSpec((2, R, D), lambda i: (0, i, 0))
_m_spec = pl.BlockSpec((R, D), lambda i: (i, 0))
_w_spec = pl.BlockSpec((D, D), lambda i: (0, 0))
_b_spec = pl.BlockSpec((1, D), lambda i: (0, 0))
_m_shape = jax.ShapeDtypeStruct((N, D), jnp.float32)


def _t0_body(deg_ref, x_ref, g0_ref):
    g0_ref[...] = x_ref[...] * _dinv_of(deg_ref)


_t0 = pl.pallas_call(
    _t0_body, grid=(N // R,),
    in_specs=[_deg_spec, _m_spec],
    out_specs=_m_spec, out_shape=_m_shape)


def _t1_body(deg_ref, p_ref, x_ref, h1_ref, g1_ref):
    dinv = _dinv_of(deg_ref)
    h1 = (p_ref[0] + p_ref[1]) * dinv + x_ref[...]
    h1_ref[...] = h1
    g1_ref[...] = h1 * dinv


_t1 = pl.pallas_call(
    _t1_body, grid=(N // R,),
    in_specs=[_deg_spec, _p_spec, _m_spec],
    out_specs=(_m_spec, _m_spec), out_shape=(_m_shape, _m_shape))


def _t2_body(deg_ref, p_ref, h1_ref, h_ref):
    h_ref[...] = (p_ref[0] + p_ref[1]) * _dinv_of(deg_ref) + h1_ref[...]


_t2 = pl.pallas_call(
    _t2_body, grid=(N // R,),
    in_specs=[_deg_spec, _p_spec, _m_spec],
    out_specs=_m_spec, out_shape=_m_shape)


def _t3_body(deg_ref, p_ref, h_ref, w1l_ref, b1_ref, w1r_ref, h2_ref):
    mean1 = (p_ref[0] + p_ref[1]) * _dninv_of(deg_ref)
    z = _matt(mean1, w1l_ref[...]) + b1_ref[...] + _matt(h_ref[...],
                                                         w1r_ref[...])
    h2_ref[...] = _SELU_SCALE * jnp.where(
        z > 0.0, z, _SELU_ALPHA * (jnp.exp(z) - 1.0))


_t3 = pl.pallas_call(
    _t3_body, grid=(N // R,),
    in_specs=[_deg_spec, _p_spec, _m_spec, _w_spec, _b_spec, _w_spec],
    out_specs=_m_spec, out_shape=_m_shape)


def _t4_body(deg_ref, p_ref, h2_ref, w2l_ref, b2_ref, w2r_ref, out_ref):
    mean2 = (p_ref[0] + p_ref[1]) * _dninv_of(deg_ref)
    z = _matt(mean2, w2l_ref[...]) + b2_ref[...] + _matt(h2_ref[...],
                                                         w2r_ref[...])
    z = z - jnp.max(z, axis=1, keepdims=True)
    ez = jnp.exp(z)
    out_ref[...] = ez / jnp.sum(ez, axis=1, keepdims=True)


_t4 = pl.pallas_call(
    _t4_body, grid=(N // R,),
    in_specs=[_deg_spec, _p_spec, _m_spec, _w_spec, _b_spec, _w_spec],
    out_specs=_m_spec, out_shape=_m_shape)


def kernel(x, edge_index, W1_l, b1, W1_r, W2_l, b2, W2_r):
    sd = (edge_index[0] | (edge_index[1] << 16)).reshape(NT, EPT)
    dstd = edge_index[1].reshape(NT, NCH, ECH)
    b1r = b1.reshape(1, D)
    b2r = b2.reshape(1, D)

    def spmm(xin):
        return _spmm_sc(sd, xin).reshape(2, N, D)

    degt = _deg_sc(dstd)[:, :, :, 0].reshape(2, N).T
    g0 = _t0(degt, x)
    p1 = spmm(g0)
    h1, g1 = _t1(degt, p1, x)
    p2 = spmm(g1)
    h = _t2(degt, p2, h1)
    p3 = spmm(h)
    h2 = _t3(degt, p3, h, W1_l, b1r, W1_r)
    p4 = spmm(h2)
    out = _t4(degt, p4, h2, W2_l, b2r, W2_r)
    return out


# speedup vs baseline: 1.0598x; 1.0188x over previous
import functools

import jax
import jax.numpy as jnp
from jax import lax
from jax.experimental import pallas as pl
from jax.experimental.pallas import tpu as pltpu
from jax.experimental.pallas import tpu_sc as plsc

N = 10000
D = 128
NT = 32
EPT = 10000
NCH = 125
ECH = 80

RPT = N // 16
DEGW = 128

_MESH = plsc.VectorSubcoreMesh(core_axis_name="c", subcore_axis_name="s")


@functools.partial(
    pl.kernel,
    mesh=_MESH,
    out_type=jax.ShapeDtypeStruct((2, 16, RPT, DEGW), jnp.float32),
    scratch_types=[
        pltpu.VMEM((NCH, ECH), jnp.int32),
        pltpu.VMEM((ECH, DEGW), jnp.float32),
        pltpu.VMEM((25, DEGW), jnp.float32),
        pltpu.VMEM_SHARED((N, DEGW), jnp.float32),
        pltpu.SemaphoreType.DMA,
        pltpu.SemaphoreType.DMA,
    ],
)
def _deg_sc(dst_hbm, out_hbm, dst_v, ones_v, zbuf, acc, sa, sb):
    c = lax.axis_index("c")
    s = lax.axis_index("s")
    tid = s * 2 + c
    one16 = jnp.ones((16,), jnp.float32)
    zero16 = jnp.zeros((16,), jnp.float32)

    def fill(i, _):
        for k in range(DEGW // 16):
            ones_v[i, pl.ds(k * 16, 16)] = one16
        return 0

    lax.fori_loop(0, ECH, fill, 0)

    def zrow(i, _):
        for k in range(DEGW // 16):
            zbuf[i, pl.ds(k * 16, 16)] = zero16
        return 0

    lax.fori_loop(0, 25, zrow, 0)
    for j in range(RPT // 25):
        pltpu.sync_copy(zbuf, acc.at[pl.ds(s * RPT + j * 25, 25)])
    pltpu.sync_copy(dst_hbm.at[tid], dst_v)
    plsc.subcore_barrier()

    pltpu.async_copy(ones_v, acc.at[dst_v.at[0]], sa, add=True)

    def chunk(i, _):
        ci = 2 * i
        pltpu.async_copy(ones_v, acc.at[dst_v.at[ci + 1]], sb, add=True)

        @pl.when(i > 0)
        def _():
            pltpu.make_async_copy(ones_v, acc.at[dst_v.at[0]], sa).wait()

        @pl.when(i < NCH // 2 - 1)
        def _():
            pltpu.async_copy(ones_v, acc.at[dst_v.at[ci + 2]], sa, add=True)
        pltpu.make_async_copy(ones_v, acc.at[dst_v.at[0]], sb).wait()
        return 0

    lax.fori_loop(0, NCH // 2, chunk, 0)
    pltpu.async_copy(ones_v, acc.at[dst_v.at[NCH - 1]], sb, add=True)
    pltpu.make_async_copy(ones_v, acc.at[dst_v.at[0]], sa).wait()
    pltpu.make_async_copy(ones_v, acc.at[dst_v.at[0]], sb).wait()
    plsc.subcore_barrier()
    pltpu.sync_copy(acc.at[pl.ds(s * RPT, RPT)], out_hbm.at[c].at[s])


@functools.partial(
    pl.kernel,
    mesh=_MESH,
    out_type=jax.ShapeDtypeStruct((2, 16, RPT, D), jnp.float32),
    scratch_types=[
        pltpu.VMEM((EPT,), jnp.int32),
        pltpu.VMEM((ECH,), jnp.int32),
        pltpu.VMEM((ECH,), jnp.int32),
        pltpu.VMEM((ECH,), jnp.int32),
        pltpu.VMEM((ECH,), jnp.int32),
        pltpu.VMEM((ECH,), jnp.int32),
        pltpu.VMEM((ECH,), jnp.int32),
        pltpu.VMEM((ECH, D), jnp.float32),
        pltpu.VMEM((ECH, D), jnp.float32),
        pltpu.VMEM((ECH, D), jnp.float32),
        pltpu.VMEM((25, D), jnp.float32),
        pltpu.VMEM_SHARED((N, D), jnp.float32),
        pltpu.SemaphoreType.DMA,
        pltpu.SemaphoreType.DMA,
        pltpu.SemaphoreType.DMA,
        pltpu.SemaphoreType.DMA,
        pltpu.SemaphoreType.DMA,
        pltpu.SemaphoreType.DMA,
    ],
)
def _spmm_sc(sd_hbm, x_hbm, out_hbm, sd_v,
             s0, s1, s2, d0, d1, d2,
             r0, r1, r2, zbuf, acc,
             ga0, ga1, ga2, sa0, sa1, sa2):
    c = lax.axis_index("c")
    s = lax.axis_index("s")
    tid = s * 2 + c
    zero16 = jnp.zeros((16,), jnp.float32)

    def zrow(i, _):
        for k in range(D // 16):
            zbuf[i, pl.ds(k * 16, 16)] = zero16
        return 0

    lax.fori_loop(0, 25, zrow, 0)
    for j in range(RPT // 25):
        pltpu.sync_copy(zbuf, acc.at[pl.ds(s * RPT + j * 25, 25)])
    pltpu.sync_copy(sd_hbm.at[tid], sd_v)
    plsc.subcore_barrier()

    SB = (s0, s1, s2)
    DB = (d0, d1, d2)
    RW = (r0, r1, r2)
    GS = (ga0, ga1, ga2)
    SS = (sa0, sa1, sa2)

    def unpack(ci, u):
        for k in range(ECH // 16):
            v = sd_v[pl.ds(ci * ECH + k * 16, 16)]
            SB[u][pl.ds(k * 16, 16)] = jnp.bitwise_and(v, 0xFFFF)
            DB[u][pl.ds(k * 16, 16)] = lax.shift_right_logical(v, 16)

    def gstart(u):
        pltpu.async_copy(x_hbm.at[SB[u]], RW[u], GS[u])

    def gwait(u):
        pltpu.make_async_copy(x_hbm.at[SB[u]], RW[u], GS[u]).wait()

    def sstart(u):
        pltpu.async_copy(RW[u], acc.at[DB[u]], SS[u], add=True)

    def swait(u):
        pltpu.make_async_copy(RW[u], acc.at[DB[u]], SS[u]).wait()

    unpack(0, 0)
    gstart(0)
    unpack(1, 1)
    gstart(1)
    NQ = 41

    def triple(i, _):
        for u in range(3):
            t = 3 * i + u
            pu = (u + 2) % 3

            if u == 0:
                @pl.when(i > 0)
                def _():
                    swait(pu)
            else:
                swait(pu)

            unpack(t + 2, pu)
            gstart(pu)
            gwait(u)
            sstart(u)
        return 0

    lax.fori_loop(0, NQ, triple, 0)
    gwait(0)
    sstart(0)
    gwait(1)
    sstart(1)
    swait(2)
    swait(0)
    swait(1)
    plsc.subcore_barrier()
    pltpu.sync_copy(acc.at[pl.ds(s * RPT, RPT)], out_hbm.at[c].at[s])


R = 2000
_SELU_ALPHA = 1.6732632423543772
_SELU_SCALE = 1.0507009873554805


def _dinv_of(deg_ref):
    dg = deg_ref[:, 0:1] + deg_ref[:, 1:2]
    return jnp.where(dg > 0.0, lax.rsqrt(dg), 0.0)


def _dninv_of(deg_ref):
    dg = deg_ref[:, 0:1] + deg_ref[:, 1:2]
    return 1.0 / jnp.maximum(dg, 1.0)


def _matt(a, w):
    return lax.dot_general(a, w, (((1,), (1,)), ((), ())),
                           preferred_element_type=jnp.float32)


_deg_spec = pl.BlockSpec((R, 2), lambda i: (i, 0))
_p_spec = pl.BlockSpec((2, R, D), lambda i: (0, i, 0))
_m_spec = pl.BlockSpec((R, D), lambda i: (i, 0))
_w_spec = pl.BlockSpec((D, D), lambda i: (0, 0))
_b_spec = pl.BlockSpec((1, D), lambda i: (0, 0))
_m_shape = jax.ShapeDtypeStruct((N, D), jnp.float32)


def _t0_body(deg_ref, x_ref, g0_ref):
    g0_ref[...] = x_ref[...] * _dinv_of(deg_ref)


_t0 = pl.pallas_call(
    _t0_body, grid=(N // R,),
    in_specs=[_deg_spec, _m_spec],
    out_specs=_m_spec, out_shape=_m_shape)


def _t1_body(deg_ref, p_ref, x_ref, h1_ref, g1_ref):
    dinv = _dinv_of(deg_ref)
    h1 = (p_ref[0] + p_ref[1]) * dinv + x_ref[...]
    h1_ref[...] = h1
    g1_ref[...] = h1 * dinv


_t1 = pl.pallas_call(
    _t1_body, grid=(N // R,),
    in_specs=[_deg_spec, _p_spec, _m_spec],
    out_specs=(_m_spec, _m_spec), out_shape=(_m_shape, _m_shape))


def _t2_body(deg_ref, p_ref, h1_ref, h_ref):
    h_ref[...] = (p_ref[0] + p_ref[1]) * _dinv_of(deg_ref) + h1_ref[...]


_t2 = pl.pallas_call(
    _t2_body, grid=(N // R,),
    in_specs=[_deg_spec, _p_spec, _m_spec],
    out_specs=_m_spec, out_shape=_m_shape)


def _t3_body(deg_ref, p_ref, h_ref, w1l_ref, b1_ref, w1r_ref, h2_ref):
    mean1 = (p_ref[0] + p_ref[1]) * _dninv_of(deg_ref)
    z = _matt(mean1, w1l_ref[...]) + b1_ref[...] + _matt(h_ref[...],
                                                         w1r_ref[...])
    h2_ref[...] = _SELU_SCALE * jnp.where(
        z > 0.0, z, _SELU_ALPHA * (jnp.exp(z) - 1.0))


_t3 = pl.pallas_call(
    _t3_body, grid=(N // R,),
    in_specs=[_deg_spec, _p_spec, _m_spec, _w_spec, _b_spec, _w_spec],
    out_specs=_m_spec, out_shape=_m_shape)


def _t4_body(deg_ref, p_ref, h2_ref, w2l_ref, b2_ref, w2r_ref, out_ref):
    mean2 = (p_ref[0] + p_ref[1]) * _dninv_of(deg_ref)
    z = _matt(mean2, w2l_ref[...]) + b2_ref[...] + _matt(h2_ref[...],
                                                         w2r_ref[...])
    z = z - jnp.max(z, axis=1, keepdims=True)
    ez = jnp.exp(z)
    out_ref[...] = ez / jnp.sum(ez, axis=1, keepdims=True)


_t4 = pl.pallas_call(
    _t4_body, grid=(N // R,),
    in_specs=[_deg_spec, _p_spec, _m_spec, _w_spec, _b_spec, _w_spec],
    out_specs=_m_spec, out_shape=_m_shape)


def kernel(x, edge_index, W1_l, b1, W1_r, W2_l, b2, W2_r):
    sd = (edge_index[0] | (edge_index[1] << 16)).reshape(NT, EPT)
    dstd = edge_index[1].reshape(NT, NCH, ECH)
    b1r = b1.reshape(1, D)
    b2r = b2.reshape(1, D)

    def spmm(xin):
        return _spmm_sc(sd, xin).reshape(2, N, D)

    degt = _deg_sc(dstd)[:, :, :, 0].reshape(2, N).T
    g0 = _t0(degt, x)
    p1 = spmm(g0)
    h1, g1 = _t1(degt, p1, x)
    p2 = spmm(g1)
    h = _t2(degt, p2, h1)
    p3 = spmm(h)
    h2 = _t3(degt, p3, h, W1_l, b1r, W1_r)
    p4 = spmm(h2)
    out = _t4(degt, p4, h2, W2_l, b2r, W2_r)
    return out
